# baseline (speedup 1.0000x reference)
import functools

import jax
import jax.numpy as jnp
from jax import lax
from jax.experimental import pallas as pl
from jax.experimental.pallas import tpu as pltpu
from jax.experimental.pallas import tpu_sc as plsc

B = 4096
EMB = 64
LANES = 128

_info = plsc.get_sparse_core_info()
_NC, _NS = _info.num_cores, _info.num_subcores
_NW = _NC * _NS
_BPW = B // _NW

_G = 4
_NB = 2 * _G
_NT = _BPW // _G

_sc_mesh = plsc.VectorSubcoreMesh(core_axis_name="c", subcore_axis_name="s")


@functools.partial(
    pl.kernel,
    mesh=_sc_mesh,
    out_type=jax.ShapeDtypeStruct((B, EMB), jnp.float32),
    scratch_types=[
        pltpu.VMEM((_BPW,), jnp.int32),
        pltpu.VMEM((_NB, EMB, LANES), jnp.float32),
        pltpu.VMEM((_BPW, EMB), jnp.float32),
        pltpu.SemaphoreType.DMA,
        pltpu.SemaphoreType.DMA,
    ],
    compiler_params=pltpu.CompilerParams(needs_layout_passes=False),
)
def _sc_gather_emb(embT_hbm, idx_hbm, w_out, idx_v, bufs, w_v, sem_a, sem_b):
    wid = lax.axis_index("s") * _NC + lax.axis_index("c")
    base = wid * _BPW
    pltpu.sync_copy(idx_hbm.at[pl.ds(base, _BPW)], idx_v)
    sems = (sem_a, sem_b)
    iota16 = jax.lax.iota(jnp.int32, 16)

    def fire_batch(t):
        g = t % 2
        vec = idx_v[pl.ds((t * _G // 16) * 16, 16)]
        for i in range(_G):
            k = t * _G + i
            r = vec[k % 16]
            off = pl.multiple_of((r >> 7) * LANES, LANES)
            pltpu.make_async_copy(
                embT_hbm.at[:, pl.ds(off, LANES)],
                bufs.at[g * _G + i],
                sems[g],
            ).start()

    def extract_batch(t):
        g = t % 2
        vec = idx_v[pl.ds((t * _G // 16) * 16, 16)]
        for i in range(_G):
            pltpu.make_async_copy(
                embT_hbm.at[:, pl.ds(0, LANES)], bufs.at[g * _G + i], sems[g]
            ).wait()
        for i in range(_G):
            k = t * _G + i
            m = vec[k % 16] & (LANES - 1)
            slot = jnp.full((16,), g * _G + i, jnp.int32)
            mv = jnp.full((16,), m, jnp.int32)
            for cc in range(EMB // 16):
                vals = plsc.load_gather(bufs, [slot, iota16 + (16 * cc), mv])
                w_v[k, pl.ds(16 * cc, 16)] = vals

    fire_batch(0)
    for t in range(1, _NT):
        fire_batch(t)
        extract_batch(t - 1)
    extract_batch(_NT - 1)

    pltpu.sync_copy(w_v, w_out.at[pl.ds(base, _BPW)])


_CH = 16


@functools.partial(
    pl.kernel,
    mesh=_sc_mesh,
    out_type=jax.ShapeDtypeStruct((B, 1), jnp.float32),
    scratch_types=[
        pltpu.VMEM((_BPW,), jnp.int32),
        pltpu.VMEM((_BPW, 1), jnp.float32),
        pltpu.SemaphoreType.DMA,
    ],
    compiler_params=pltpu.CompilerParams(use_tc_tiling_on_sc=False),
)
def _sc_gather_bias(bias_hbm, idx_hbm, b_out, idx_v, bias_v, sem):
    wid = lax.axis_index("s") * _NC + lax.axis_index("c")
    base = wid * _BPW
    pltpu.sync_copy(idx_hbm.at[pl.ds(base, _BPW)], idx_v)
    for ch in range(_BPW // _CH):
        vec = idx_v[pl.ds(ch * _CH, _CH)]
        for j in range(_CH):
            k = ch * _CH + j
            r = vec[j]
            pltpu.make_async_copy(
                bias_hbm.at[pl.ds(r, 1), :], bias_v.at[pl.ds(k, 1), :], sem
            ).start()
        pltpu.make_async_copy(
            bias_hbm.at[pl.ds(0, _CH), :], bias_v.at[pl.ds(0, _CH), :], sem
        ).wait()
    pltpu.sync_copy(bias_v, b_out.at[pl.ds(base, _BPW)])


_TM = 1024
_TN = 1024
_NI = B // _TM
_NJ = B // _TN


def _loss_body(w_i_ref, wT_j_ref, bcol_ref, brow_ref, lx_ref, wgt_ref, out_ref):
    i = pl.program_id(0)
    j = pl.program_id(1)
    t = jnp.dot(w_i_ref[...], wT_j_ref[...], preferred_element_type=jnp.float32)
    d = t + bcol_ref[...] + brow_ref[...] - lx_ref[...]
    s = jnp.sum(wgt_ref[...] * d * d).reshape(1, 1)

    is_first = (i == 0) & (j == 0)
    is_last = (i == _NI - 1) & (j == _NJ - 1)

    @pl.when(is_first)
    def _():
        out_ref[...] = s

    @pl.when(jnp.logical_not(is_first))
    def _():
        out_ref[...] = out_ref[...] + s

    @pl.when(is_last)
    def _():
        out_ref[...] = out_ref[...] * (1.0 / (B * B))


def _tc_loss(w, wT, b_col, b_row, log_x, weights):
    return pl.pallas_call(
        _loss_body,
        grid=(_NI, _NJ),
        in_specs=[
            pl.BlockSpec((_TM, EMB), lambda i, j: (i, 0)),
            pl.BlockSpec((EMB, _TN), lambda i, j: (0, j)),
            pl.BlockSpec((_TM, 1), lambda i, j: (i, 0)),
            pl.BlockSpec((1, _TN), lambda i, j: (0, j)),
            pl.BlockSpec((_TM, _TN), lambda i, j: (i, j)),
            pl.BlockSpec((_TM, _TN), lambda i, j: (i, j)),
        ],
        out_specs=pl.BlockSpec((1, 1), lambda i, j: (0, 0)),
        out_shape=jax.ShapeDtypeStruct((1, 1), jnp.float32),
    )(w, wT, b_col, b_row, log_x, weights)


def kernel(indices, log_x, weights, emb_table, bias_table):
    idx = indices.astype(jnp.int32)
    w = _sc_gather_emb(emb_table.T, idx)
    b = _sc_gather_bias(bias_table, idx)
    loss = _tc_loss(w, w.T, b, b.reshape(1, B), log_x, weights)
    return loss[0, 0]


# speedup vs baseline: 5.6683x; 5.6683x over previous
import functools

import jax
import jax.numpy as jnp
from jax import lax
from jax.experimental import pallas as pl
from jax.experimental.pallas import tpu as pltpu
from jax.experimental.pallas import tpu_sc as plsc

B = 4096
EMB = 64
LANES = 128

_info = plsc.get_sparse_core_info()
_NC, _NS = _info.num_cores, _info.num_subcores
_NW = _NC * _NS
_BPW = B // _NW

_G = 4
_NB = 2 * _G
_NT = _BPW // _G

_sc_mesh = plsc.VectorSubcoreMesh(core_axis_name="c", subcore_axis_name="s")


@functools.partial(
    pl.kernel,
    mesh=_sc_mesh,
    out_type=jax.ShapeDtypeStruct((B, EMB), jnp.float32),
    scratch_types=[
        pltpu.VMEM((_BPW,), jnp.int32),
        pltpu.VMEM((_NB, EMB, LANES), jnp.float32),
        pltpu.VMEM((_BPW, EMB), jnp.float32),
        pltpu.SemaphoreType.DMA,
        pltpu.SemaphoreType.DMA,
    ],
    compiler_params=pltpu.CompilerParams(needs_layout_passes=False),
)
def _sc_gather_emb(embT_hbm, idx_hbm, w_out, idx_v, bufs, w_v, sem_a, sem_b):
    wid = lax.axis_index("s") * _NC + lax.axis_index("c")
    base = wid * _BPW
    pltpu.sync_copy(idx_hbm.at[pl.ds(base, _BPW)], idx_v)
    sems = (sem_a, sem_b)
    iota16 = jax.lax.iota(jnp.int32, 16)

    def fire_batch(t):
        g = t % 2
        vec = idx_v[pl.ds((t * _G // 16) * 16, 16)]
        for i in range(_G):
            k = t * _G + i
            r = vec[k % 16]
            off = pl.multiple_of((r >> 7) * LANES, LANES)
            pltpu.make_async_copy(
                embT_hbm.at[:, pl.ds(off, LANES)],
                bufs.at[g * _G + i],
                sems[g],
            ).start()

    def extract_batch(t):
        g = t % 2
        vec = idx_v[pl.ds((t * _G // 16) * 16, 16)]
        for i in range(_G):
            pltpu.make_async_copy(
                embT_hbm.at[:, pl.ds(0, LANES)], bufs.at[g * _G + i], sems[g]
            ).wait()
        for i in range(_G):
            k = t * _G + i
            m = vec[k % 16] & (LANES - 1)
            slot = jnp.full((16,), g * _G + i, jnp.int32)
            mv = jnp.full((16,), m, jnp.int32)
            for cc in range(EMB // 16):
                vals = plsc.load_gather(bufs, [slot, iota16 + (16 * cc), mv])
                w_v[k, pl.ds(16 * cc, 16)] = vals

    fire_batch(0)
    for t in range(1, _NT):
        fire_batch(t)
        extract_batch(t - 1)
    extract_batch(_NT - 1)

    pltpu.sync_copy(w_v, w_out.at[pl.ds(base, _BPW)])


_CH = 16


@functools.partial(
    pl.kernel,
    mesh=_sc_mesh,
    out_type=jax.ShapeDtypeStruct((B,), jnp.float32),
    scratch_types=[
        pltpu.VMEM((_BPW,), jnp.int32),
        pltpu.VMEM((_BPW,), jnp.float32),
        pltpu.SemaphoreType.DMA,
    ],
    compiler_params=pltpu.CompilerParams(use_tc_tiling_on_sc=False),
)
def _sc_gather_bias(bias_hbm, idx_hbm, b_out, idx_v, bias_v, sem):
    wid = lax.axis_index("s") * _NC + lax.axis_index("c")
    base = wid * _BPW
    pltpu.sync_copy(idx_hbm.at[pl.ds(base, _BPW)], idx_v)
    pltpu.async_copy(bias_hbm.at[idx_v], bias_v, sem).wait()
    pltpu.sync_copy(bias_v, b_out.at[pl.ds(base, _BPW)])


_TM = 1024
_TN = 1024
_NI = B // _TM
_NJ = B // _TN


def _loss_body(w_i_ref, wT_j_ref, bcol_ref, brow_ref, lx_ref, wgt_ref, out_ref):
    i = pl.program_id(0)
    j = pl.program_id(1)
    t = jnp.dot(w_i_ref[...], wT_j_ref[...], preferred_element_type=jnp.float32)
    d = t + bcol_ref[...] + brow_ref[...] - lx_ref[...]
    s = jnp.sum(wgt_ref[...] * d * d).reshape(1, 1)

    is_first = (i == 0) & (j == 0)
    is_last = (i == _NI - 1) & (j == _NJ - 1)

    @pl.when(is_first)
    def _():
        out_ref[...] = s

    @pl.when(jnp.logical_not(is_first))
    def _():
        out_ref[...] = out_ref[...] + s

    @pl.when(is_last)
    def _():
        out_ref[...] = out_ref[...] * (1.0 / (B * B))


def _tc_loss(w, wT, b_col, b_row, log_x, weights):
    return pl.pallas_call(
        _loss_body,
        grid=(_NI, _NJ),
        in_specs=[
            pl.BlockSpec((_TM, EMB), lambda i, j: (i, 0)),
            pl.BlockSpec((EMB, _TN), lambda i, j: (0, j)),
            pl.BlockSpec((_TM, 1), lambda i, j: (i, 0)),
            pl.BlockSpec((1, _TN), lambda i, j: (0, j)),
            pl.BlockSpec((_TM, _TN), lambda i, j: (i, j)),
            pl.BlockSpec((_TM, _TN), lambda i, j: (i, j)),
        ],
        out_specs=pl.BlockSpec((1, 1), lambda i, j: (0, 0)),
        out_shape=jax.ShapeDtypeStruct((1, 1), jnp.float32),
    )(w, wT, b_col, b_row, log_x, weights)


def kernel(indices, log_x, weights, emb_table, bias_table):
    idx = indices.astype(jnp.int32)
    w = _sc_gather_emb(emb_table.T, idx)
    b = _sc_gather_bias(bias_table.reshape(-1), idx)
    loss = _tc_loss(w, w.T, b.reshape(B, 1), b.reshape(1, B), log_x, weights)
    return loss[0, 0]


# speedup vs baseline: 5.8252x; 1.0277x over previous
import functools

import jax
import jax.numpy as jnp
from jax import lax
from jax.experimental import pallas as pl
from jax.experimental.pallas import tpu as pltpu
from jax.experimental.pallas import tpu_sc as plsc

B = 4096
EMB = 64
LANES = 128

_info = plsc.get_sparse_core_info()
_NC, _NS = _info.num_cores, _info.num_subcores
_NW = _NC * _NS
_BPW = B // _NW

_G = 4
_NG = 3
_NB = _NG * _G
_NT = _BPW // _G

_sc_mesh = plsc.VectorSubcoreMesh(core_axis_name="c", subcore_axis_name="s")


@functools.partial(
    pl.kernel,
    mesh=_sc_mesh,
    out_type=jax.ShapeDtypeStruct((B, EMB), jnp.float32),
    scratch_types=[
        pltpu.VMEM((_BPW,), jnp.int32),
        pltpu.VMEM((_NB, EMB, LANES), jnp.float32),
        pltpu.VMEM((_BPW, EMB), jnp.float32),
        pltpu.SemaphoreType.DMA,
        pltpu.SemaphoreType.DMA,
        pltpu.SemaphoreType.DMA,
    ],
    compiler_params=pltpu.CompilerParams(needs_layout_passes=False),
)
def _sc_gather_emb(embT_hbm, idx_hbm, w_out, idx_v, bufs, w_v,
                   sem_a, sem_b, sem_c):
    wid = lax.axis_index("s") * _NC + lax.axis_index("c")
    base = wid * _BPW
    pltpu.sync_copy(idx_hbm.at[pl.ds(base, _BPW)], idx_v)
    sems = (sem_a, sem_b, sem_c)
    iota16 = jax.lax.iota(jnp.int32, 16)

    def fire_batch(t):
        g = t % _NG
        vec = idx_v[pl.ds((t * _G // 16) * 16, 16)]
        for i in range(_G):
            k = t * _G + i
            r = vec[k % 16]
            off = pl.multiple_of((r >> 7) * LANES, LANES)
            pltpu.make_async_copy(
                embT_hbm.at[:, pl.ds(off, LANES)],
                bufs.at[g * _G + i],
                sems[g],
            ).start()

    def extract_batch(t):
        g = t % _NG
        vec = idx_v[pl.ds((t * _G // 16) * 16, 16)]
        for i in range(_G):
            pltpu.make_async_copy(
                embT_hbm.at[:, pl.ds(0, LANES)], bufs.at[g * _G + i], sems[g]
            ).wait()
        for i in range(_G):
            k = t * _G + i
            m = vec[k % 16] & (LANES - 1)
            slot = jnp.full((16,), g * _G + i, jnp.int32)
            mv = jnp.full((16,), m, jnp.int32)
            for cc in range(EMB // 16):
                vals = plsc.load_gather(bufs, [slot, iota16 + (16 * cc), mv])
                w_v[k, pl.ds(16 * cc, 16)] = vals

    for t in range(_NG - 1):
        fire_batch(t)
    for t in range(_NG - 1, _NT):
        fire_batch(t)
        extract_batch(t - (_NG - 1))
    for t in range(_NT - (_NG - 1), _NT):
        extract_batch(t)

    pltpu.sync_copy(w_v, w_out.at[pl.ds(base, _BPW)])


_CH = 16


@functools.partial(
    pl.kernel,
    mesh=_sc_mesh,
    out_type=jax.ShapeDtypeStruct((B,), jnp.float32),
    scratch_types=[
        pltpu.VMEM((_BPW,), jnp.int32),
        pltpu.VMEM((_BPW,), jnp.float32),
        pltpu.SemaphoreType.DMA,
    ],
    compiler_params=pltpu.CompilerParams(use_tc_tiling_on_sc=False),
)
def _sc_gather_bias(bias_hbm, idx_hbm, b_out, idx_v, bias_v, sem):
    wid = lax.axis_index("s") * _NC + lax.axis_index("c")
    base = wid * _BPW
    pltpu.sync_copy(idx_hbm.at[pl.ds(base, _BPW)], idx_v)
    pltpu.async_copy(bias_hbm.at[idx_v], bias_v, sem).wait()
    pltpu.sync_copy(bias_v, b_out.at[pl.ds(base, _BPW)])


_TM = 1024
_TN = 1024
_NI = B // _TM
_NJ = B // _TN


def _loss_body(w_i_ref, wT_j_ref, bcol_ref, brow_ref, lx_ref, wgt_ref, out_ref):
    i = pl.program_id(0)
    j = pl.program_id(1)
    t = jnp.dot(w_i_ref[...], wT_j_ref[...], preferred_element_type=jnp.float32)
    d = t + bcol_ref[...] + brow_ref[...] - lx_ref[...]
    s = jnp.sum(wgt_ref[...] * d * d).reshape(1, 1)

    is_first = (i == 0) & (j == 0)
    is_last = (i == _NI - 1) & (j == _NJ - 1)

    @pl.when(is_first)
    def _():
        out_ref[...] = s

    @pl.when(jnp.logical_not(is_first))
    def _():
        out_ref[...] = out_ref[...] + s

    @pl.when(is_last)
    def _():
        out_ref[...] = out_ref[...] * (1.0 / (B * B))


def _tc_loss(w, wT, b_col, b_row, log_x, weights):
    return pl.pallas_call(
        _loss_body,
        grid=(_NI, _NJ),
        in_specs=[
            pl.BlockSpec((_TM, EMB), lambda i, j: (i, 0)),
            pl.BlockSpec((EMB, _TN), lambda i, j: (0, j)),
            pl.BlockSpec((_TM, 1), lambda i, j: (i, 0)),
            pl.BlockSpec((1, _TN), lambda i, j: (0, j)),
            pl.BlockSpec((_TM, _TN), lambda i, j: (i, j)),
            pl.BlockSpec((_TM, _TN), lambda i, j: (i, j)),
        ],
        out_specs=pl.BlockSpec((1, 1), lambda i, j: (0, 0)),
        out_shape=jax.ShapeDtypeStruct((1, 1), jnp.float32),
    )(w, wT, b_col, b_row, log_x, weights)


def kernel(indices, log_x, weights, emb_table, bias_table):
    idx = indices.astype(jnp.int32)
    w = _sc_gather_emb(emb_table.T, idx)
    b = _sc_gather_bias(bias_table.reshape(-1), idx)
    loss = _tc_loss(w, w.T, b.reshape(B, 1), b.reshape(1, B), log_x, weights)
    return loss[0, 0]
